# baseline (speedup 1.0000x reference)
import functools

import jax
import jax.numpy as jnp
from jax import lax
from jax.experimental import pallas as pl
from jax.experimental.pallas import tpu as pltpu
from jax.experimental.pallas import tpu_sc as plsc

N = 10000
E = 160000
DF = 256
DH = 512
DC = 40

_ROWS = 400
_GRID = N // _ROWS
_NSUB = 16
_NPAD = 10240
_RPS = _NPAD // _NSUB
_ZR = 128


def _vmesh():
    return plsc.VectorSubcoreMesh(core_axis_name="c", subcore_axis_name="s")


_KF = 80
_EPS_E = 5200
_EPAD = 32 * _EPS_E
_TRASH = 10100


def _make_deg():
    K = _KF
    NCH = _EPS_E // K

    @functools.partial(
        pl.kernel,
        out_type=jax.ShapeDtypeStruct((2, _NPAD, 16), jnp.float32),
        mesh=_vmesh(),
        scratch_types=[
            pltpu.VMEM((NCH, K), jnp.int32),
            pltpu.VMEM((K, 16), jnp.float32),
            pltpu.VMEM((_ZR, 16), jnp.float32),
            pltpu.VMEM_SHARED((_NPAD, 16), jnp.float32),
        ],
    )
    def deg_kernel(dst2, out, didx2, ones, zbuf, acc):
        cid = lax.axis_index("c")
        sid = lax.axis_index("s")

        @pl.loop(0, _ZR)
        def _(r):
            zbuf[r, :] = jnp.zeros((16,), jnp.float32)

        @pl.loop(0, K)
        def _(r):
            ones[r, :] = jnp.full((16,), 1.0, jnp.float32)

        pltpu.sync_copy(dst2.at[cid * _NSUB + sid], didx2)

        @pl.loop(0, _RPS // _ZR)
        def _(j):
            pltpu.sync_copy(zbuf, acc.at[pl.ds(sid * _RPS + j * _ZR, _ZR), :])

        plsc.subcore_barrier()

        @pl.loop(0, NCH)
        def _(c):
            pltpu.sync_copy(ones, acc.at[didx2.at[c]], add=True)

        plsc.subcore_barrier()

        @pl.loop(0, _RPS // _ZR)
        def _(j):
            r0 = sid * _RPS + j * _ZR
            pltpu.sync_copy(acc.at[pl.ds(r0, _ZR), :], out.at[cid].at[pl.ds(r0, _ZR), :])

    return deg_kernel


def _edge_pipeline(gather_src, sidx1, didx2, k, msg0, msg1, sems, acc, nch):
    assert nch % 2 == 1 and nch >= 5
    gs0, gs1, ss0, ss1 = sems

    def g_start(c, msg, sem):
        off = pl.multiple_of(c * k, 8)
        pltpu.make_async_copy(gather_src.at[sidx1.at[pl.ds(off, k)]], msg,
                              sem).start()

    def g_wait(msg, sem):
        pltpu.make_async_copy(gather_src.at[sidx1.at[pl.ds(0, k)]], msg,
                              sem).wait()

    def s_start(c, msg, sem):
        pltpu.make_async_copy(msg, acc.at[didx2.at[c]], sem).start(add=True)

    def s_wait(msg, sem):
        pltpu.make_async_copy(msg, acc.at[didx2.at[0]], sem).wait()

    g_start(0, msg0, gs0)
    g_start(1, msg1, gs1)

    @pl.loop(0, (nch - 3) // 2)
    def _(t):
        c0 = 2 * t
        g_wait(msg0, gs0)
        s_start(c0, msg0, ss0)
        g_wait(msg1, gs1)
        s_start(c0 + 1, msg1, ss1)
        s_wait(msg0, ss0)
        g_start(c0 + 2, msg0, gs0)
        s_wait(msg1, ss1)
        g_start(c0 + 3, msg1, gs1)

    g_wait(msg0, gs0)
    s_start(nch - 3, msg0, ss0)
    g_wait(msg1, gs1)
    s_start(nch - 2, msg1, ss1)
    s_wait(msg0, ss0)
    g_start(nch - 1, msg0, gs0)
    g_wait(msg0, gs0)
    s_start(nch - 1, msg0, ss0)
    s_wait(msg1, ss1)
    s_wait(msg0, ss0)


def _zero_acc(msg0, acc, sid, k, dc):
    @pl.loop(0, k)
    def _(r):
        @pl.loop(0, dc, step=16)
        def _(cc):
            msg0[r, pl.ds(cc, 16)] = jnp.zeros((16,), jnp.float32)

    @pl.loop(0, _RPS // k)
    def _(j):
        pltpu.sync_copy(msg0, acc.at[pl.ds(sid * _RPS + j * k, k), :])


def _make_prop(nb, dc):
    K = _KF
    EPS = E // _NSUB
    NCH = EPS // K
    bpc = nb // 2

    @functools.partial(
        pl.kernel,
        out_type=jax.ShapeDtypeStruct((nb, _NPAD, dc), jnp.float32),
        mesh=_vmesh(),
        scratch_types=[
            pltpu.VMEM((EPS,), jnp.int32),
            pltpu.VMEM((NCH, K), jnp.int32),
            pltpu.VMEM((K, dc), jnp.float32),
            pltpu.VMEM((K, dc), jnp.float32),
            pltpu.VMEM_SHARED((_NPAD, dc), jnp.float32),
            pltpu.SemaphoreType.DMA,
            pltpu.SemaphoreType.DMA,
            pltpu.SemaphoreType.DMA,
            pltpu.SemaphoreType.DMA,
        ],
    )
    def prop_kernel(xsb, src, dst2, out,
                    sidx1, didx2, msg0, msg1, acc, gs0, gs1, ss0, ss1):
        cid = lax.axis_index("c")
        sid = lax.axis_index("s")

        pltpu.sync_copy(src.at[pl.ds(sid * EPS, EPS)], sidx1)
        pltpu.sync_copy(dst2.at[sid], didx2)

        for b in range(nb):
            @pl.when(cid == b // bpc)
            def _():
                _zero_acc(msg0, acc, sid, K, dc)
                plsc.subcore_barrier()

                _edge_pipeline(xsb.at[b], sidx1, didx2, K,
                               msg0, msg1, (gs0, gs1, ss0, ss1), acc, NCH)

                plsc.subcore_barrier()

                @pl.loop(0, _RPS // _ZR)
                def _(j):
                    r0 = sid * _RPS + j * _ZR
                    pltpu.sync_copy(acc.at[pl.ds(r0, _ZR), :],
                                    out.at[b].at[pl.ds(r0, _ZR), :])

    return prop_kernel


def _make_prop_esplit():
    K = _KF
    EPS = _EPS_E
    NCH = EPS // K
    dc = 128

    @functools.partial(
        pl.kernel,
        out_type=jax.ShapeDtypeStruct((2, _NPAD, dc), jnp.float32),
        mesh=_vmesh(),
        scratch_types=[
            pltpu.VMEM((EPS,), jnp.int32),
            pltpu.VMEM((NCH, K), jnp.int32),
            pltpu.VMEM((K, dc), jnp.float32),
            pltpu.VMEM((K, dc), jnp.float32),
            pltpu.VMEM_SHARED((_NPAD, dc), jnp.float32),
            pltpu.SemaphoreType.DMA,
            pltpu.SemaphoreType.DMA,
            pltpu.SemaphoreType.DMA,
            pltpu.SemaphoreType.DMA,
        ],
    )
    def prop_kernel(xs, src, dst2, out,
                    sidx1, didx2, msg0, msg1, acc, gs0, gs1, ss0, ss1):
        cid = lax.axis_index("c")
        sid = lax.axis_index("s")

        wid = cid * _NSUB + sid
        pltpu.sync_copy(src.at[pl.ds(wid * EPS, EPS)], sidx1)
        pltpu.sync_copy(dst2.at[wid], didx2)

        _zero_acc(msg0, acc, sid, K, dc)
        plsc.subcore_barrier()

        _edge_pipeline(xs, sidx1, didx2, K, msg0, msg1,
                       (gs0, gs1, ss0, ss1), acc, NCH)

        plsc.subcore_barrier()

        @pl.loop(0, _RPS // _ZR)
        def _(j):
            r0 = sid * _RPS + j * _ZR
            pltpu.sync_copy(acc.at[pl.ds(r0, _ZR), :],
                            out.at[cid].at[pl.ds(r0, _ZR), :])

    return prop_kernel


def _prep_call(degp, x):
    def body(degp_ref, x_ref, dinv_ref, xsb_ref):
        deg = degp_ref[0, :, 0:1] + degp_ref[1, :, 0:1] + 1.0
        dinv = lax.rsqrt(deg)
        dinv_ref[...] = dinv
        xs = x_ref[...] * dinv
        xsb_ref[0] = xs[:, :128]
        xsb_ref[1] = xs[:, 128:]

    return pl.pallas_call(
        body,
        grid=(_GRID,),
        in_specs=[
            pl.BlockSpec((2, _ROWS, 16), lambda i: (0, i, 0)),
            pl.BlockSpec((_ROWS, DF), lambda i: (i, 0)),
        ],
        out_specs=[
            pl.BlockSpec((_ROWS, 1), lambda i: (i, 0)),
            pl.BlockSpec((2, _ROWS, 128), lambda i: (0, i, 0)),
        ],
        out_shape=[
            jax.ShapeDtypeStruct((N, 1), jnp.float32),
            jax.ShapeDtypeStruct((2, N, 128), jnp.float32),
        ],
    )(degp, x)


def _layer1_call(acc1, x, dinv, W1, b1):
    def body(acc_ref, x_ref, dinv_ref, W1_ref, b1_ref, h1_ref, hs_ref):
        d = dinv_ref[...]
        acc = jnp.concatenate([acc_ref[0], acc_ref[1]], axis=1)
        p0 = d * acc + (d * d) * x_ref[...]
        h1 = jnp.maximum(
            jnp.dot(p0, W1_ref[...], preferred_element_type=jnp.float32)
            + b1_ref[...], 0.0)
        h1_ref[...] = h1
        hs = h1 * d
        for b in range(4):
            hs_ref[b] = hs[:, b * 128:(b + 1) * 128]

    return pl.pallas_call(
        body,
        grid=(_GRID,),
        in_specs=[
            pl.BlockSpec((2, _ROWS, 128), lambda i: (0, i, 0)),
            pl.BlockSpec((_ROWS, DF), lambda i: (i, 0)),
            pl.BlockSpec((_ROWS, 1), lambda i: (i, 0)),
            pl.BlockSpec((DF, DH), lambda i: (0, 0)),
            pl.BlockSpec((1, DH), lambda i: (0, 0)),
        ],
        out_specs=[
            pl.BlockSpec((_ROWS, DH), lambda i: (i, 0)),
            pl.BlockSpec((4, _ROWS, 128), lambda i: (0, i, 0)),
        ],
        out_shape=[
            jax.ShapeDtypeStruct((N, DH), jnp.float32),
            jax.ShapeDtypeStruct((4, N, 128), jnp.float32),
        ],
    )(acc1, x, dinv, W1, b1)


def _layer2_call(acc2, h1, dinv, W2, b2, Wl):
    def body(acc_ref, h1_ref, dinv_ref, W2_ref, b2_ref, Wl_ref, y_ref, ys_ref):
        d = dinv_ref[...]
        acc = jnp.concatenate([acc_ref[b] for b in range(4)], axis=1)
        h1 = h1_ref[...]
        p1 = d * acc + (d * d) * h1
        h2 = jnp.maximum(
            jnp.dot(p1, W2_ref[...], preferred_element_type=jnp.float32)
            + b2_ref[...], 0.0)
        wl = Wl_ref[...]
        y = (jnp.dot(h1, wl[:DH], preferred_element_type=jnp.float32)
             + jnp.dot(h2, wl[DH:], preferred_element_type=jnp.float32))
        y_ref[...] = y
        ys = y * d
        ys_ref[...] = jnp.concatenate(
            [ys, jnp.zeros((_ROWS, 128 - DC), jnp.float32)], axis=1)

    return pl.pallas_call(
        body,
        grid=(_GRID,),
        in_specs=[
            pl.BlockSpec((4, _ROWS, 128), lambda i: (0, i, 0)),
            pl.BlockSpec((_ROWS, DH), lambda i: (i, 0)),
            pl.BlockSpec((_ROWS, 1), lambda i: (i, 0)),
            pl.BlockSpec((DH, DH), lambda i: (0, 0)),
            pl.BlockSpec((1, DH), lambda i: (0, 0)),
            pl.BlockSpec((2 * DH, DC), lambda i: (0, 0)),
        ],
        out_specs=[
            pl.BlockSpec((_ROWS, DC), lambda i: (i, 0)),
            pl.BlockSpec((_ROWS, 128), lambda i: (i, 0)),
        ],
        out_shape=[
            jax.ShapeDtypeStruct((N, DC), jnp.float32),
            jax.ShapeDtypeStruct((N, 128), jnp.float32),
        ],
    )(acc2, h1, dinv, W2, b2, Wl)


def _final_call(acc3, y, dinv, bl):
    def body(acc_ref, y_ref, dinv_ref, bl_ref, out_ref):
        d = dinv_ref[...]
        acc40 = (acc_ref[0] + acc_ref[1])[:, :DC]
        out_ref[...] = d * acc40 + (d * d) * y_ref[...] + bl_ref[...]

    return pl.pallas_call(
        body,
        grid=(_GRID,),
        in_specs=[
            pl.BlockSpec((2, _ROWS, 128), lambda i: (0, i, 0)),
            pl.BlockSpec((_ROWS, DC), lambda i: (i, 0)),
            pl.BlockSpec((_ROWS, 1), lambda i: (i, 0)),
            pl.BlockSpec((1, DC), lambda i: (0, 0)),
        ],
        out_specs=pl.BlockSpec((_ROWS, DC), lambda i: (i, 0)),
        out_shape=jax.ShapeDtypeStruct((N, DC), jnp.float32),
    )(acc3, y, dinv, bl)


_DEG = _make_deg()
_PROP1 = _make_prop(2, 128)
_PROP2 = _make_prop(4, 128)
_PROP3 = _make_prop_esplit()


def kernel(x, edge_index, W1, b1, W2, b2, Wl, bl):
    ei = edge_index.astype(jnp.int32)
    src, dst = ei[0], ei[1]
    dst_f = dst.reshape(_NSUB, (E // _NSUB) // _KF, _KF)
    srcp = jnp.concatenate([src, jnp.zeros((_EPAD - E,), jnp.int32)])
    dstp = jnp.concatenate([dst, jnp.full((_EPAD - E,), _TRASH, jnp.int32)])
    dst_e = dstp.reshape(2 * _NSUB, _EPS_E // _KF, _KF)
    degp = _DEG(dst_e)
    dinv, xsb = _prep_call(degp, x)
    acc1 = _PROP1(xsb, src, dst_f)
    h1, hs1 = _layer1_call(acc1, x, dinv, W1, b1.reshape(1, DH))
    acc2 = _PROP2(hs1, src, dst_f)
    y, ysb = _layer2_call(acc2, h1, dinv, W2, b2.reshape(1, DH), Wl)
    acc3 = _PROP3(ysb, srcp, dst_e)
    out = _final_call(acc3, y, dinv, bl.reshape(1, DC))
    return (out, out)


# speedup vs baseline: 1.0024x; 1.0024x over previous
import functools

import jax
import jax.numpy as jnp
from jax import lax
from jax.experimental import pallas as pl
from jax.experimental.pallas import tpu as pltpu
from jax.experimental.pallas import tpu_sc as plsc

N = 10000
E = 160000
DF = 256
DH = 512
DC = 40

_ROWS = 400
_GRID = N // _ROWS
_NSUB = 16
_NPAD = 10240
_RPS = _NPAD // _NSUB
_ZR = 128


def _vmesh():
    return plsc.VectorSubcoreMesh(core_axis_name="c", subcore_axis_name="s")


_KF = 80
_EPS_E = 5200
_EPAD = 32 * _EPS_E
_TRASH = 10100


def _make_deg():
    K = _KF
    NCH = _EPS_E // K

    @functools.partial(
        pl.kernel,
        out_type=jax.ShapeDtypeStruct((2, _NPAD, 16), jnp.float32),
        mesh=_vmesh(),
        scratch_types=[
            pltpu.VMEM((NCH, K), jnp.int32),
            pltpu.VMEM((K, 16), jnp.float32),
            pltpu.VMEM((_ZR, 16), jnp.float32),
            pltpu.VMEM_SHARED((_NPAD, 16), jnp.float32),
        ],
    )
    def deg_kernel(dst2, out, didx2, ones, zbuf, acc):
        cid = lax.axis_index("c")
        sid = lax.axis_index("s")

        @pl.loop(0, _ZR)
        def _(r):
            zbuf[r, :] = jnp.zeros((16,), jnp.float32)

        @pl.loop(0, K)
        def _(r):
            ones[r, :] = jnp.full((16,), 1.0, jnp.float32)

        pltpu.sync_copy(dst2.at[cid * _NSUB + sid], didx2)

        @pl.loop(0, _RPS // _ZR)
        def _(j):
            pltpu.sync_copy(zbuf, acc.at[pl.ds(sid * _RPS + j * _ZR, _ZR), :])

        plsc.subcore_barrier()

        @pl.loop(0, NCH)
        def _(c):
            pltpu.sync_copy(ones, acc.at[didx2.at[c]], add=True)

        plsc.subcore_barrier()

        @pl.loop(0, _RPS // _ZR)
        def _(j):
            r0 = sid * _RPS + j * _ZR
            pltpu.sync_copy(acc.at[pl.ds(r0, _ZR), :], out.at[cid].at[pl.ds(r0, _ZR), :])

    return deg_kernel


def _edge_pipeline(gather_src, sidx1, didx2, k, msg0, msg1, sems, acc, nch):
    assert nch % 2 == 1 and nch >= 5
    gs0, gs1, ss0, ss1 = sems

    def g_start(c, msg, sem):
        off = pl.multiple_of(c * k, 8)
        pltpu.make_async_copy(gather_src.at[sidx1.at[pl.ds(off, k)]], msg,
                              sem).start()

    def g_wait(msg, sem):
        pltpu.make_async_copy(gather_src.at[sidx1.at[pl.ds(0, k)]], msg,
                              sem).wait()

    def s_start(c, msg, sem):
        pltpu.make_async_copy(msg, acc.at[didx2.at[c]], sem).start(add=True)

    def s_wait(msg, sem):
        pltpu.make_async_copy(msg, acc.at[didx2.at[0]], sem).wait()

    g_start(0, msg0, gs0)
    g_start(1, msg1, gs1)

    @pl.loop(0, (nch - 3) // 2)
    def _(t):
        c0 = 2 * t
        g_wait(msg0, gs0)
        s_start(c0, msg0, ss0)
        g_wait(msg1, gs1)
        s_start(c0 + 1, msg1, ss1)
        s_wait(msg0, ss0)
        g_start(c0 + 2, msg0, gs0)
        s_wait(msg1, ss1)
        g_start(c0 + 3, msg1, gs1)

    g_wait(msg0, gs0)
    s_start(nch - 3, msg0, ss0)
    g_wait(msg1, gs1)
    s_start(nch - 2, msg1, ss1)
    s_wait(msg0, ss0)
    g_start(nch - 1, msg0, gs0)
    g_wait(msg0, gs0)
    s_start(nch - 1, msg0, ss0)
    s_wait(msg1, ss1)
    s_wait(msg0, ss0)


def _zero_acc(msg0, acc, sid, k, dc):
    @pl.loop(0, k)
    def _(r):
        @pl.loop(0, dc, step=16)
        def _(cc):
            msg0[r, pl.ds(cc, 16)] = jnp.zeros((16,), jnp.float32)

    @pl.loop(0, _RPS // k)
    def _(j):
        pltpu.sync_copy(msg0, acc.at[pl.ds(sid * _RPS + j * k, k), :])


def _make_prop(nb, dc):
    K = _KF
    EPS = E // _NSUB
    NCH = EPS // K
    bpc = nb // 2

    @functools.partial(
        pl.kernel,
        out_type=jax.ShapeDtypeStruct((nb, _NPAD, dc), jnp.float32),
        mesh=_vmesh(),
        scratch_types=[
            pltpu.VMEM((EPS,), jnp.int32),
            pltpu.VMEM((NCH, K), jnp.int32),
            pltpu.VMEM((K, dc), jnp.float32),
            pltpu.VMEM((K, dc), jnp.float32),
            pltpu.VMEM_SHARED((_NPAD, dc), jnp.float32),
            pltpu.SemaphoreType.DMA,
            pltpu.SemaphoreType.DMA,
            pltpu.SemaphoreType.DMA,
            pltpu.SemaphoreType.DMA,
        ],
    )
    def prop_kernel(xsb, src, dst2, out,
                    sidx1, didx2, msg0, msg1, acc, gs0, gs1, ss0, ss1):
        cid = lax.axis_index("c")
        sid = lax.axis_index("s")

        pltpu.sync_copy(src.at[pl.ds(sid * EPS, EPS)], sidx1)
        pltpu.sync_copy(dst2.at[sid], didx2)

        for b in range(nb):
            @pl.when(cid == b // bpc)
            def _():
                _zero_acc(msg0, acc, sid, K, dc)
                plsc.subcore_barrier()

                _edge_pipeline(xsb.at[b], sidx1, didx2, K,
                               msg0, msg1, (gs0, gs1, ss0, ss1), acc, NCH)

                plsc.subcore_barrier()

                @pl.loop(0, _RPS // _ZR)
                def _(j):
                    r0 = sid * _RPS + j * _ZR
                    pltpu.sync_copy(acc.at[pl.ds(r0, _ZR), :],
                                    out.at[b].at[pl.ds(r0, _ZR), :])

    return prop_kernel


def _make_prop_esplit():
    K = _KF
    EPS = _EPS_E
    NCH = EPS // K
    dc = 128

    @functools.partial(
        pl.kernel,
        out_type=jax.ShapeDtypeStruct((2, _NPAD, dc), jnp.float32),
        mesh=_vmesh(),
        scratch_types=[
            pltpu.VMEM((EPS,), jnp.int32),
            pltpu.VMEM((NCH, K), jnp.int32),
            pltpu.VMEM((K, dc), jnp.float32),
            pltpu.VMEM((K, dc), jnp.float32),
            pltpu.VMEM_SHARED((_NPAD, dc), jnp.float32),
            pltpu.SemaphoreType.DMA,
            pltpu.SemaphoreType.DMA,
            pltpu.SemaphoreType.DMA,
            pltpu.SemaphoreType.DMA,
        ],
    )
    def prop_kernel(xs, src, dst2, out,
                    sidx1, didx2, msg0, msg1, acc, gs0, gs1, ss0, ss1):
        cid = lax.axis_index("c")
        sid = lax.axis_index("s")

        wid = cid * _NSUB + sid
        pltpu.sync_copy(src.at[pl.ds(wid * EPS, EPS)], sidx1)
        pltpu.sync_copy(dst2.at[wid], didx2)

        _zero_acc(msg0, acc, sid, K, dc)
        plsc.subcore_barrier()

        _edge_pipeline(xs, sidx1, didx2, K, msg0, msg1,
                       (gs0, gs1, ss0, ss1), acc, NCH)

        plsc.subcore_barrier()

        @pl.loop(0, _RPS // _ZR)
        def _(j):
            r0 = sid * _RPS + j * _ZR
            pltpu.sync_copy(acc.at[pl.ds(r0, _ZR), :],
                            out.at[cid].at[pl.ds(r0, _ZR), :])

    return prop_kernel


def _prep_call(degp, x):
    def body(degp_ref, x_ref, dinv_ref, xsb_ref):
        deg = degp_ref[0, :, 0:1] + degp_ref[1, :, 0:1] + 1.0
        dinv = lax.rsqrt(deg)
        dinv_ref[...] = dinv
        xs = x_ref[...] * dinv
        xsb_ref[0] = xs[:, :128]
        xsb_ref[1] = xs[:, 128:]

    return pl.pallas_call(
        body,
        grid=(_GRID,),
        in_specs=[
            pl.BlockSpec((2, _ROWS, 16), lambda i: (0, i, 0)),
            pl.BlockSpec((_ROWS, DF), lambda i: (i, 0)),
        ],
        out_specs=[
            pl.BlockSpec((_ROWS, 1), lambda i: (i, 0)),
            pl.BlockSpec((2, _ROWS, 128), lambda i: (0, i, 0)),
        ],
        out_shape=[
            jax.ShapeDtypeStruct((N, 1), jnp.float32),
            jax.ShapeDtypeStruct((2, N, 128), jnp.float32),
        ],
    )(degp, x)


def _layer1_call(acc1, x, dinv, W1, b1):
    def body(acc_ref, x_ref, dinv_ref, W1_ref, b1_ref, h1_ref, hs_ref):
        d = dinv_ref[...]
        acc = jnp.concatenate([acc_ref[0], acc_ref[1]], axis=1)
        p0 = d * acc + (d * d) * x_ref[...]
        h1 = jnp.maximum(
            jnp.dot(p0, W1_ref[...], preferred_element_type=jnp.float32)
            + b1_ref[...], 0.0)
        h1_ref[...] = h1
        hs = h1 * d
        for b in range(4):
            hs_ref[b] = hs[:, b * 128:(b + 1) * 128]

    return pl.pallas_call(
        body,
        grid=(_GRID,),
        in_specs=[
            pl.BlockSpec((2, _ROWS, 128), lambda i: (0, i, 0)),
            pl.BlockSpec((_ROWS, DF), lambda i: (i, 0)),
            pl.BlockSpec((_ROWS, 1), lambda i: (i, 0)),
            pl.BlockSpec((DF, DH), lambda i: (0, 0)),
            pl.BlockSpec((1, DH), lambda i: (0, 0)),
        ],
        out_specs=[
            pl.BlockSpec((_ROWS, DH), lambda i: (i, 0)),
            pl.BlockSpec((4, _ROWS, 128), lambda i: (0, i, 0)),
        ],
        out_shape=[
            jax.ShapeDtypeStruct((N, DH), jnp.float32),
            jax.ShapeDtypeStruct((4, N, 128), jnp.float32),
        ],
    )(acc1, x, dinv, W1, b1)


def _layer2_call(acc2, h1, dinv, W2, b2, Wl):
    def body(acc_ref, h1_ref, dinv_ref, W2_ref, b2_ref, Wl_ref, y_ref, ys_ref):
        d = dinv_ref[...]
        acc = jnp.concatenate([acc_ref[b] for b in range(4)], axis=1)
        h1 = h1_ref[...]
        p1 = d * acc + (d * d) * h1
        h2 = jnp.maximum(
            jnp.dot(p1, W2_ref[...], preferred_element_type=jnp.float32)
            + b2_ref[...], 0.0)
        wl = Wl_ref[...]
        y = (jnp.dot(h1, wl[:DH], preferred_element_type=jnp.float32)
             + jnp.dot(h2, wl[DH:], preferred_element_type=jnp.float32))
        y_ref[...] = y
        ys = y * d
        ys_ref[...] = jnp.concatenate(
            [ys, jnp.zeros((_ROWS, 128 - DC), jnp.float32)], axis=1)

    return pl.pallas_call(
        body,
        grid=(_GRID,),
        in_specs=[
            pl.BlockSpec((4, _ROWS, 128), lambda i: (0, i, 0)),
            pl.BlockSpec((_ROWS, DH), lambda i: (i, 0)),
            pl.BlockSpec((_ROWS, 1), lambda i: (i, 0)),
            pl.BlockSpec((DH, DH), lambda i: (0, 0)),
            pl.BlockSpec((1, DH), lambda i: (0, 0)),
            pl.BlockSpec((2 * DH, DC), lambda i: (0, 0)),
        ],
        out_specs=[
            pl.BlockSpec((_ROWS, DC), lambda i: (i, 0)),
            pl.BlockSpec((_ROWS, 128), lambda i: (i, 0)),
        ],
        out_shape=[
            jax.ShapeDtypeStruct((N, DC), jnp.float32),
            jax.ShapeDtypeStruct((N, 128), jnp.float32),
        ],
    )(acc2, h1, dinv, W2, b2, Wl)


def _final_call(acc3, y, dinv, bl):
    def body(acc_ref, y_ref, dinv_ref, bl_ref, out_ref):
        d = dinv_ref[...]
        acc40 = (acc_ref[0] + acc_ref[1])[:, :DC]
        out_ref[...] = d * acc40 + (d * d) * y_ref[...] + bl_ref[...]

    return pl.pallas_call(
        body,
        grid=(_GRID,),
        in_specs=[
            pl.BlockSpec((2, _ROWS, 128), lambda i: (0, i, 0)),
            pl.BlockSpec((_ROWS, DC), lambda i: (i, 0)),
            pl.BlockSpec((_ROWS, 1), lambda i: (i, 0)),
            pl.BlockSpec((1, DC), lambda i: (0, 0)),
        ],
        out_specs=pl.BlockSpec((_ROWS, DC), lambda i: (i, 0)),
        out_shape=jax.ShapeDtypeStruct((N, DC), jnp.float32),
    )(acc3, y, dinv, bl)


_DEG = _make_deg()
_PROP1 = _make_prop(2, 128)
_PROP2 = _make_prop(4, 128)
_PROP3 = _make_prop_esplit()


def kernel(x, edge_index, W1, b1, W2, b2, Wl, bl):
    ei = edge_index.astype(jnp.int32)
    src, dst = ei[0], ei[1]
    dst_f = dst.reshape(_NSUB, (E // _NSUB) // _KF, _KF)
    srcp = jnp.concatenate([src, jnp.zeros((_EPAD - E,), jnp.int32)])
    trash = N + (jnp.arange(_EPAD - E, dtype=jnp.int32) % (_NPAD - N - 8))
    dstp = jnp.concatenate([dst, trash])
    dst_e = dstp.reshape(2 * _NSUB, _EPS_E // _KF, _KF)
    degp = _DEG(dst_e)
    dinv, xsb = _prep_call(degp, x)
    acc1 = _PROP1(xsb, src, dst_f)
    h1, hs1 = _layer1_call(acc1, x, dinv, W1, b1.reshape(1, DH))
    acc2 = _PROP2(hs1, src, dst_f)
    y, ysb = _layer2_call(acc2, h1, dinv, W2, b2.reshape(1, DH), Wl)
    acc3 = _PROP3(ysb, srcp, dst_e)
    out = _final_call(acc3, y, dinv, bl.reshape(1, DC))
    return (out, out)


# speedup vs baseline: 1.3029x; 1.2997x over previous
import functools

import jax
import jax.numpy as jnp
from jax import lax
from jax.experimental import pallas as pl
from jax.experimental.pallas import tpu as pltpu
from jax.experimental.pallas import tpu_sc as plsc

N = 10000
E = 160000
DF = 256
DH = 512
DC = 40

_ROWS = 400
_GRID = N // _ROWS
_NSUB = 16
_NPAD = 10240
_RPS = _NPAD // _NSUB
_ZR = 128


def _vmesh():
    return plsc.VectorSubcoreMesh(core_axis_name="c", subcore_axis_name="s")


_KF = 80
_EPS_E = 5200
_EPAD = 32 * _EPS_E
_TRASH = 10100


def _make_deg():
    K = _KF
    NCH = _EPS_E // K

    @functools.partial(
        pl.kernel,
        out_type=jax.ShapeDtypeStruct((2, _NPAD, 16), jnp.float32),
        mesh=_vmesh(),
        scratch_types=[
            pltpu.VMEM((NCH, K), jnp.int32),
            pltpu.VMEM((K, 16), jnp.float32),
            pltpu.VMEM((_ZR, 16), jnp.float32),
            pltpu.VMEM_SHARED((_NPAD, 16), jnp.float32),
        ],
    )
    def deg_kernel(dst2, out, didx2, ones, zbuf, acc):
        cid = lax.axis_index("c")
        sid = lax.axis_index("s")

        @pl.loop(0, _ZR)
        def _(r):
            zbuf[r, :] = jnp.zeros((16,), jnp.float32)

        @pl.loop(0, K)
        def _(r):
            ones[r, :] = jnp.full((16,), 1.0, jnp.float32)

        pltpu.sync_copy(dst2.at[cid * _NSUB + sid], didx2)

        @pl.loop(0, _RPS // _ZR)
        def _(j):
            pltpu.sync_copy(zbuf, acc.at[pl.ds(sid * _RPS + j * _ZR, _ZR), :])

        plsc.subcore_barrier()

        @pl.loop(0, NCH)
        def _(c):
            pltpu.sync_copy(ones, acc.at[didx2.at[c]], add=True)

        plsc.subcore_barrier()

        @pl.loop(0, _RPS // _ZR)
        def _(j):
            r0 = sid * _RPS + j * _ZR
            pltpu.sync_copy(acc.at[pl.ds(r0, _ZR), :], out.at[cid].at[pl.ds(r0, _ZR), :])

    return deg_kernel


def _edge_pipeline(gather_src, sidx1, didx2, k, msg0, msg1, sems, acc, nch):
    assert nch % 2 == 1 and nch >= 5
    gs0, gs1, ss0, ss1 = sems

    def g_start(c, msg, sem):
        off = pl.multiple_of(c * k, 8)
        pltpu.make_async_copy(gather_src.at[sidx1.at[pl.ds(off, k)]], msg,
                              sem).start()

    def g_wait(msg, sem):
        pltpu.make_async_copy(gather_src.at[sidx1.at[pl.ds(0, k)]], msg,
                              sem).wait()

    def s_start(c, msg, sem):
        pltpu.make_async_copy(msg, acc.at[didx2.at[c]], sem).start(add=True)

    def s_wait(msg, sem):
        pltpu.make_async_copy(msg, acc.at[didx2.at[0]], sem).wait()

    g_start(0, msg0, gs0)
    g_start(1, msg1, gs1)

    @pl.loop(0, (nch - 3) // 2)
    def _(t):
        c0 = 2 * t
        g_wait(msg0, gs0)
        s_start(c0, msg0, ss0)
        g_wait(msg1, gs1)
        s_start(c0 + 1, msg1, ss1)
        s_wait(msg0, ss0)
        g_start(c0 + 2, msg0, gs0)
        s_wait(msg1, ss1)
        g_start(c0 + 3, msg1, gs1)

    g_wait(msg0, gs0)
    s_start(nch - 3, msg0, ss0)
    g_wait(msg1, gs1)
    s_start(nch - 2, msg1, ss1)
    s_wait(msg0, ss0)
    g_start(nch - 1, msg0, gs0)
    g_wait(msg0, gs0)
    s_start(nch - 1, msg0, ss0)
    s_wait(msg1, ss1)
    s_wait(msg0, ss0)


def _zero_acc(msg0, acc, sid, k, dc):
    @pl.loop(0, k)
    def _(r):
        @pl.loop(0, dc, step=16)
        def _(cc):
            msg0[r, pl.ds(cc, 16)] = jnp.zeros((16,), jnp.float32)

    @pl.loop(0, _RPS // k)
    def _(j):
        pltpu.sync_copy(msg0, acc.at[pl.ds(sid * _RPS + j * k, k), :])


def _make_prop(nb, dc):
    K = _KF
    EPS = E // _NSUB
    NCH = EPS // K
    bpc = nb // 2

    @functools.partial(
        pl.kernel,
        out_type=jax.ShapeDtypeStruct((nb, _NPAD, dc), jnp.float32),
        mesh=_vmesh(),
        scratch_types=[
            pltpu.VMEM((EPS,), jnp.int32),
            pltpu.VMEM((NCH, K), jnp.int32),
            pltpu.VMEM((K, dc), jnp.float32),
            pltpu.VMEM((K, dc), jnp.float32),
            pltpu.VMEM_SHARED((_NPAD, dc), jnp.float32),
            pltpu.SemaphoreType.DMA,
            pltpu.SemaphoreType.DMA,
            pltpu.SemaphoreType.DMA,
            pltpu.SemaphoreType.DMA,
        ],
    )
    def prop_kernel(xsb, src, dst2, out,
                    sidx1, didx2, msg0, msg1, acc, gs0, gs1, ss0, ss1):
        cid = lax.axis_index("c")
        sid = lax.axis_index("s")

        pltpu.sync_copy(src.at[pl.ds(sid * EPS, EPS)], sidx1)
        pltpu.sync_copy(dst2.at[sid], didx2)

        for b in range(nb):
            @pl.when(cid == b // bpc)
            def _():
                _zero_acc(msg0, acc, sid, K, dc)
                plsc.subcore_barrier()

                _edge_pipeline(xsb.at[b], sidx1, didx2, K,
                               msg0, msg1, (gs0, gs1, ss0, ss1), acc, NCH)

                plsc.subcore_barrier()

                @pl.loop(0, _RPS // _ZR)
                def _(j):
                    r0 = sid * _RPS + j * _ZR
                    pltpu.sync_copy(acc.at[pl.ds(r0, _ZR), :],
                                    out.at[b].at[pl.ds(r0, _ZR), :])

    return prop_kernel


def _make_prop_esplit():
    K = _KF
    EPS = _EPS_E
    NCH = EPS // K
    dc = 128

    @functools.partial(
        pl.kernel,
        out_type=jax.ShapeDtypeStruct((2, _NPAD, dc), jnp.float32),
        mesh=_vmesh(),
        scratch_types=[
            pltpu.VMEM((EPS,), jnp.int32),
            pltpu.VMEM((NCH, K), jnp.int32),
            pltpu.VMEM((K, dc), jnp.float32),
            pltpu.VMEM((K, dc), jnp.float32),
            pltpu.VMEM_SHARED((_NPAD, dc), jnp.float32),
            pltpu.SemaphoreType.DMA,
            pltpu.SemaphoreType.DMA,
            pltpu.SemaphoreType.DMA,
            pltpu.SemaphoreType.DMA,
        ],
    )
    def prop_kernel(xs, src, dst2, out,
                    sidx1, didx2, msg0, msg1, acc, gs0, gs1, ss0, ss1):
        cid = lax.axis_index("c")
        sid = lax.axis_index("s")

        wid = cid * _NSUB + sid
        pltpu.sync_copy(src.at[pl.ds(wid * EPS, EPS)], sidx1)
        pltpu.sync_copy(dst2.at[wid], didx2)

        _zero_acc(msg0, acc, sid, K, dc)
        plsc.subcore_barrier()

        _edge_pipeline(xs, sidx1, didx2, K, msg0, msg1,
                       (gs0, gs1, ss0, ss1), acc, NCH)

        plsc.subcore_barrier()

        @pl.loop(0, _RPS // _ZR)
        def _(j):
            r0 = sid * _RPS + j * _ZR
            pltpu.sync_copy(acc.at[pl.ds(r0, _ZR), :],
                            out.at[cid].at[pl.ds(r0, _ZR), :])

    return prop_kernel


def _prep_call(degp, x):
    def body(degp_ref, x_ref, dinv_ref, xsb_ref):
        deg = degp_ref[0, :, 0:1] + degp_ref[1, :, 0:1] + 1.0
        dinv = lax.rsqrt(deg)
        dinv_ref[...] = dinv
        xs = x_ref[...] * dinv
        xsb_ref[0] = xs[:, :128]
        xsb_ref[1] = xs[:, 128:]

    return pl.pallas_call(
        body,
        grid=(_GRID,),
        in_specs=[
            pl.BlockSpec((2, _ROWS, 16), lambda i: (0, i, 0)),
            pl.BlockSpec((_ROWS, DF), lambda i: (i, 0)),
        ],
        out_specs=[
            pl.BlockSpec((_ROWS, 1), lambda i: (i, 0)),
            pl.BlockSpec((2, _ROWS, 128), lambda i: (0, i, 0)),
        ],
        out_shape=[
            jax.ShapeDtypeStruct((N, 1), jnp.float32),
            jax.ShapeDtypeStruct((2, N, 128), jnp.float32),
        ],
    )(degp, x)


def _layer1_call(acc1, x, dinv, W1, b1):
    def body(acc_ref, x_ref, dinv_ref, W1_ref, b1_ref, h1_ref, hs_ref):
        d = dinv_ref[...]
        acc = jnp.concatenate([acc_ref[0], acc_ref[1]], axis=1)
        p0 = d * acc + (d * d) * x_ref[...]
        h1 = jnp.maximum(
            jnp.dot(p0, W1_ref[...], preferred_element_type=jnp.float32)
            + b1_ref[...], 0.0)
        h1_ref[...] = h1
        hs = h1 * d
        for b in range(4):
            hs_ref[b] = hs[:, b * 128:(b + 1) * 128]

    return pl.pallas_call(
        body,
        grid=(_GRID,),
        in_specs=[
            pl.BlockSpec((2, _ROWS, 128), lambda i: (0, i, 0)),
            pl.BlockSpec((_ROWS, DF), lambda i: (i, 0)),
            pl.BlockSpec((_ROWS, 1), lambda i: (i, 0)),
            pl.BlockSpec((DF, DH), lambda i: (0, 0)),
            pl.BlockSpec((1, DH), lambda i: (0, 0)),
        ],
        out_specs=[
            pl.BlockSpec((_ROWS, DH), lambda i: (i, 0)),
            pl.BlockSpec((4, _ROWS, 128), lambda i: (0, i, 0)),
        ],
        out_shape=[
            jax.ShapeDtypeStruct((N, DH), jnp.float32),
            jax.ShapeDtypeStruct((4, N, 128), jnp.float32),
        ],
    )(acc1, x, dinv, W1, b1)


def _layer2_call(acc2, h1, dinv, W2, b2, Wl):
    def body(acc_ref, h1_ref, dinv_ref, W2_ref, b2_ref, Wl_ref, y_ref, ys_ref):
        d = dinv_ref[...]
        acc = jnp.concatenate([acc_ref[b] for b in range(4)], axis=1)
        h1 = h1_ref[...]
        p1 = d * acc + (d * d) * h1
        h2 = jnp.maximum(
            jnp.dot(p1, W2_ref[...], preferred_element_type=jnp.float32)
            + b2_ref[...], 0.0)
        wl = Wl_ref[...]
        y = (jnp.dot(h1, wl[:DH], preferred_element_type=jnp.float32)
             + jnp.dot(h2, wl[DH:], preferred_element_type=jnp.float32))
        y_ref[...] = y
        ys = y * d
        ys_ref[...] = jnp.concatenate(
            [ys, jnp.zeros((_ROWS, 128 - DC), jnp.float32)], axis=1)

    return pl.pallas_call(
        body,
        grid=(_GRID,),
        in_specs=[
            pl.BlockSpec((4, _ROWS, 128), lambda i: (0, i, 0)),
            pl.BlockSpec((_ROWS, DH), lambda i: (i, 0)),
            pl.BlockSpec((_ROWS, 1), lambda i: (i, 0)),
            pl.BlockSpec((DH, DH), lambda i: (0, 0)),
            pl.BlockSpec((1, DH), lambda i: (0, 0)),
            pl.BlockSpec((2 * DH, DC), lambda i: (0, 0)),
        ],
        out_specs=[
            pl.BlockSpec((_ROWS, DC), lambda i: (i, 0)),
            pl.BlockSpec((_ROWS, 128), lambda i: (i, 0)),
        ],
        out_shape=[
            jax.ShapeDtypeStruct((N, DC), jnp.float32),
            jax.ShapeDtypeStruct((N, 128), jnp.float32),
        ],
    )(acc2, h1, dinv, W2, b2, Wl)


def _final_call(acc3, y, dinv, bl):
    def body(acc_ref, y_ref, dinv_ref, bl_ref, out_ref):
        d = dinv_ref[...]
        acc40 = (acc_ref[0] + acc_ref[1])[:, :DC]
        out_ref[...] = d * acc40 + (d * d) * y_ref[...] + bl_ref[...]

    return pl.pallas_call(
        body,
        grid=(_GRID,),
        in_specs=[
            pl.BlockSpec((2, _ROWS, 128), lambda i: (0, i, 0)),
            pl.BlockSpec((_ROWS, DC), lambda i: (i, 0)),
            pl.BlockSpec((_ROWS, 1), lambda i: (i, 0)),
            pl.BlockSpec((1, DC), lambda i: (0, 0)),
        ],
        out_specs=pl.BlockSpec((_ROWS, DC), lambda i: (i, 0)),
        out_shape=jax.ShapeDtypeStruct((N, DC), jnp.float32),
    )(acc3, y, dinv, bl)


_DEG = _make_deg()
_PROP1 = _make_prop(2, 128)
_PROP2 = _make_prop(4, 128)
_PROP3 = _make_prop_esplit()


def kernel(x, edge_index, W1, b1, W2, b2, Wl, bl):
    ei = edge_index.astype(jnp.int32)
    src, dst = ei[0], ei[1]
    dst_f = dst.reshape(_NSUB, (E // _NSUB) // _KF, _KF)
    pad_i = jnp.arange(_EPAD - E, dtype=jnp.int32)
    srcp = jnp.concatenate([src, pad_i % N])
    dstp = jnp.concatenate([dst, N + pad_i % (_NPAD - N - 8)])
    dst_e = dstp.reshape(2 * _NSUB, _EPS_E // _KF, _KF)
    degp = _DEG(dst_e)
    dinv, xsb = _prep_call(degp, x)
    acc1 = _PROP1(xsb, src, dst_f)
    h1, hs1 = _layer1_call(acc1, x, dinv, W1, b1.reshape(1, DH))
    acc2 = _PROP2(hs1, src, dst_f)
    y, ysb = _layer2_call(acc2, h1, dinv, W2, b2.reshape(1, DH), Wl)
    acc3 = _PROP3(ysb, srcp, dst_e)
    out = _final_call(acc3, y, dinv, bl.reshape(1, DC))
    return (out, out)
